# Initial kernel scaffold; baseline (speedup 1.0000x reference)
import jax
import jax.numpy as jnp
from jax.experimental import pallas as pl


def kernel(x, cluster, batch, W1, b1, ln_g, ln_b, W2, b2):
    raise NotImplementedError("write your pallas kernel here")


# speedup vs baseline: 1.7437x; 1.7437x over previous
import functools

import jax
import jax.numpy as jnp
from jax import lax
from jax.experimental import pallas as pl
from jax.experimental.pallas import tpu as pltpu
from jax.experimental.pallas import tpu_sc as plsc

N = 320000
IN_DIMS = 128
HIDDEN = 256
D = IN_DIMS // 2
C = 10000

NC = 2
NS = 16
NW = NC * NS
R = N // NW

CH = 400
CAP = 1024
FLUSH_THR = CAP - CH - 1
AGGR_ROWS = C + NW


MLP_BLK = 512


def _mlp_body(x_ref, w1_ref, b1_ref, g_ref, b_ref, w2_ref, b2_ref, o_ref):
    h = jnp.dot(x_ref[...], w1_ref[...], preferred_element_type=jnp.float32)
    h = h + b1_ref[...]
    mu = jnp.mean(h, axis=1, keepdims=True)
    d = h - mu
    var = jnp.mean(d * d, axis=1, keepdims=True)
    hn = d * lax.rsqrt(var + 1e-5) * g_ref[...] + b_ref[...]
    hs = hn * (1.0 / (1.0 + jnp.exp(-hn)))
    o = jnp.dot(hs, w2_ref[...], preferred_element_type=jnp.float32)
    o_ref[...] = o + b2_ref[...]


def _mlp(x, W1, b1, ln_g, ln_b, W2, b2):
    grid = (N // MLP_BLK,)
    return pl.pallas_call(
        _mlp_body,
        grid=grid,
        in_specs=[
            pl.BlockSpec((MLP_BLK, IN_DIMS), lambda i: (i, 0)),
            pl.BlockSpec((IN_DIMS, HIDDEN), lambda i: (0, 0)),
            pl.BlockSpec((1, HIDDEN), lambda i: (0, 0)),
            pl.BlockSpec((1, HIDDEN), lambda i: (0, 0)),
            pl.BlockSpec((1, HIDDEN), lambda i: (0, 0)),
            pl.BlockSpec((HIDDEN, D), lambda i: (0, 0)),
            pl.BlockSpec((1, D), lambda i: (0, 0)),
        ],
        out_specs=pl.BlockSpec((MLP_BLK, D), lambda i: (i, 0)),
        out_shape=jax.ShapeDtypeStruct((N, D), jnp.float32),
        compiler_params=pltpu.CompilerParams(
            dimension_semantics=("arbitrary",),
        ),
    )(x, W1, b1, ln_g, ln_b, W2, b2)


_MESH = plsc.VectorSubcoreMesh(core_axis_name="c", subcore_axis_name="s")
_NEGINF = float("-inf")


@functools.partial(
    pl.kernel,
    mesh=_MESH,
    out_type=jax.ShapeDtypeStruct((AGGR_ROWS, D), jnp.float32),
    scratch_types=[
        pltpu.VMEM((CH, D), jnp.float32),
        pltpu.VMEM((CH,), jnp.int32),
        pltpu.VMEM((CAP, D), jnp.float32),
        pltpu.VMEM((CAP * 16,), jnp.int32),
        pltpu.VMEM((8, 128), jnp.int32),
        pltpu.VMEM((16,), jnp.int32),
        pltpu.VMEM((16,), jnp.int32),
        pltpu.VMEM((4, 16), jnp.float32),
        pltpu.SemaphoreType.DMA,
    ],
    compiler_params=pltpu.CompilerParams(needs_layout_passes=False,
                                         use_tc_tiling_on_sc=False),
)
def _scan(out_hbm, cl_hbm, aggr_hbm, data_v, cids_v, stage_v, ids_v, idsc_v,
          pb_v, state_v, acc_v, sem):
    wid = lax.axis_index("s") * NC + lax.axis_index("c")
    base = wid * R
    end = base + R
    dummy = jnp.int32(C) + wid
    dvec = jnp.broadcast_to(dummy, (16,))
    neg = jnp.full((16,), _NEGINF, jnp.float32)
    iota = lax.iota(jnp.int32, 16)

    def _ids_reset(s, _):
        ids_v[pl.ds(s * 16, 16)] = dvec
        return 0

    lax.fori_loop(0, CAP, _ids_reset, 0)

    @pl.when(wid > 0)
    def _():
        pltpu.sync_copy(cl_hbm.at[pl.ds(base - 16, 16)], pb_v)

    prev0 = jnp.where(wid > 0, pb_v[...][15], jnp.int32(-1))

    def _scatter(epoch, run_id):
        @pl.when(epoch == 0)
        def _():
            ids_v[pl.ds(0, 16)] = dvec
        for j in range(8):
            @pl.when(run_id >= j * 128)
            def _():
                for m in range(8):
                    g = plsc.load_gather(
                        ids_v, [(iota + (j * 128 + m * 16)) * 16])
                    idsc_v[j, pl.ds(m * 16, 16)] = g
                pltpu.async_copy(
                    stage_v.at[pl.ds(j * 128, 128)],
                    aggr_hbm.at[idsc_v.at[j]], sem).wait()

    def _flush(prev, run_id, epoch):
        fl = run_id >= FLUSH_THR

        @pl.when(fl)
        def _():
            _scatter(epoch, run_id)
            lax.fori_loop(0, CAP, _ids_reset, 0)
            for q in range(4):
                stage_v[0, pl.ds(q * 16, 16)] = stage_v[run_id,
                                                        pl.ds(q * 16, 16)]
            ids_v[pl.ds(0, 16)] = jnp.broadcast_to(prev, (16,))

        return jnp.where(fl, 0, run_id), epoch + fl.astype(jnp.int32)

    def _load_chunk(start):
        pltpu.sync_copy(out_hbm.at[pl.ds(start, CH)], data_v)
        pltpu.sync_copy(cl_hbm.at[pl.ds(start, CH)], cids_v)

    def _row_step(cid, r_slot, prev, run_id, accs):
        same = cid == prev
        run_id = run_id + (1 - same.astype(jnp.int32))
        for c in range(4):
            v = data_v[r_slot, pl.ds(c * 16, 16)]
            accs[c] = jnp.maximum(jnp.where(same, accs[c], neg), v)
            stage_v[run_id, pl.ds(c * 16, 16)] = accs[c]
        ids_v[pl.ds(run_id * 16, 16)] = jnp.broadcast_to(cid, (16,))
        return cid, run_id

    def _chunk_main(k, carry):
        prev, run_id, epoch, a0, a1, a2, a3 = carry
        run_id, epoch = _flush(prev, run_id, epoch)
        _load_chunk(base + k * CH)
        accs = [a0, a1, a2, a3]

        def _group(q, gc):
            prev, run_id, a0, a1, a2, a3 = gc
            cid_vec = cids_v[pl.ds(q * 16, 16)]
            ac = [a0, a1, a2, a3]
            for l in range(16):
                prev, run_id = _row_step(cid_vec[l], q * 16 + l, prev, run_id,
                                         ac)
            return (prev, run_id, ac[0], ac[1], ac[2], ac[3])

        prev, run_id, a0, a1, a2, a3 = lax.fori_loop(
            0, CH // 16, _group, (prev, run_id, accs[0], accs[1], accs[2],
                                  accs[3]))
        return (prev, run_id, epoch, a0, a1, a2, a3)

    prev, run_id, epoch, a0, a1, a2, a3 = lax.fori_loop(
        0, R // CH, _chunk_main, (prev0, jnp.int32(0), jnp.int32(0),
                                  neg, neg, neg, neg))

    state_v[...] = jnp.where(iota == 0, prev,
                   jnp.where(iota == 1, run_id,
                   jnp.where(iota == 2, epoch, jnp.int32(0))))
    for c, a in enumerate((a0, a1, a2, a3)):
        acc_v[c, pl.ds(0, 16)] = a

    def _chunk_ext(j, _):
        st = state_v[...]

        @pl.when(st[3] == 0)
        def _():
            prev, run_id, epoch = st[0], st[1], st[2]
            run_id, epoch = _flush(prev, run_id, epoch)
            start = end + j * CH
            _load_chunk(start)
            accs = [acc_v[c, pl.ds(0, 16)] for c in range(4)]

            def _group(q, gc):
                prev, run_id, done, a0, a1, a2, a3 = gc
                cid_vec = cids_v[pl.ds(q * 16, 16)]
                ac = [a0, a1, a2, a3]
                for l in range(16):
                    cid = cid_vec[l]
                    same = cid == prev
                    done_new = jnp.logical_or(done, jnp.logical_not(same))
                    adv = jnp.logical_and(jnp.logical_not(same),
                                          jnp.logical_not(done_new))
                    run_id = run_id + adv.astype(jnp.int32)
                    for c in range(4):
                        v = data_v[q * 16 + l, pl.ds(c * 16, 16)]
                        ac[c] = jnp.where(
                            done_new, ac[c],
                            jnp.maximum(jnp.where(same, ac[c], neg), v))
                        stage_v[run_id, pl.ds(c * 16, 16)] = ac[c]
                    idst = jnp.where(done_new, prev, cid)
                    ids_v[pl.ds(run_id * 16, 16)] = jnp.broadcast_to(idst,
                                                                     (16,))
                    prev = idst
                    done = done_new
                return (prev, run_id, done, ac[0], ac[1], ac[2], ac[3])

            prev, run_id, done, na0, na1, na2, na3 = lax.fori_loop(
                0, CH // 16, _group,
                (prev, run_id, jnp.bool_(False), accs[0], accs[1], accs[2],
                 accs[3]))
            state_v[...] = jnp.where(iota == 0, prev,
                           jnp.where(iota == 1, run_id,
                           jnp.where(iota == 2, epoch, done.astype(jnp.int32))))
            for c, a in enumerate((na0, na1, na2, na3)):
                acc_v[c, pl.ds(0, 16)] = a

        return 0

    lax.fori_loop(0, (N - end) // CH, _chunk_ext, 0)
    stf = state_v[...]
    _scatter(stf[2], stf[1])


GCH = 400
_G_PIECES = ((0, 128), (128, 128), (256, 128), (384, 16))


@functools.partial(
    pl.kernel,
    mesh=_MESH,
    out_type=jax.ShapeDtypeStruct((N, 2 * D), jnp.float32),
    scratch_types=[
        pltpu.VMEM((GCH,), jnp.int32),
        pltpu.VMEM((GCH, D), jnp.float32),
        pltpu.VMEM((GCH, D), jnp.float32),
        pltpu.SemaphoreType.DMA,
        pltpu.SemaphoreType.DMA,
    ],
    compiler_params=pltpu.CompilerParams(needs_layout_passes=False,
                                         use_tc_tiling_on_sc=False),
)
def _gather(cl_hbm, out_hbm, aggr_hbm, y_hbm, idx_v, g_v, o_v, sem, sem2):
    wid = lax.axis_index("s") * NC + lax.axis_index("c")
    base = wid * R

    def _chunk(k, _):
        start = base + k * GCH
        pltpu.sync_copy(cl_hbm.at[pl.ds(start, GCH)], idx_v)
        co = pltpu.async_copy(out_hbm.at[pl.ds(start, GCH)], o_v, sem2)
        cps = []
        for off, ln in _G_PIECES:
            cps.append(pltpu.async_copy(
                aggr_hbm.at[idx_v.at[pl.ds(off, ln)]],
                g_v.at[pl.ds(off, ln)], sem))
        for cp in cps:
            cp.wait()
        co.wait()
        pltpu.sync_copy(o_v, y_hbm.at[pl.ds(start, GCH), pl.ds(0, D)])
        pltpu.sync_copy(g_v, y_hbm.at[pl.ds(start, GCH), pl.ds(D, D)])
        return 0

    lax.fori_loop(0, R // GCH, _chunk, 0)


def kernel(x, cluster, batch, W1, b1, ln_g, ln_b, W2, b2):
    del batch
    cluster = cluster.astype(jnp.int32)
    out = _mlp(x, W1, b1.reshape(1, -1), ln_g.reshape(1, -1),
               ln_b.reshape(1, -1), W2, b2.reshape(1, -1))
    aggr = _scan(out, cluster)
    return _gather(cluster, out, aggr)


# speedup vs baseline: 1.8617x; 1.0677x over previous
import functools

import jax
import jax.numpy as jnp
from jax import lax
from jax.experimental import pallas as pl
from jax.experimental.pallas import tpu as pltpu
from jax.experimental.pallas import tpu_sc as plsc

N = 320000
IN_DIMS = 128
HIDDEN = 256
D = IN_DIMS // 2
C = 10000

NC = 2
NS = 16
NW = NC * NS
R = N // NW

CH = 400
CAP = 768
FLUSH_THR = CAP - CH - 1
AGGR_ROWS = C + NW

MLP_BLK = 512


def _mlp_body(x_ref, w1_ref, b1_ref, g_ref, b_ref, w2_ref, b2_ref, o_ref):
    xb = x_ref[...].astype(jnp.bfloat16)
    h = jnp.dot(xb, w1_ref[...], preferred_element_type=jnp.float32)
    h = h + b1_ref[...]
    mu = jnp.mean(h, axis=1, keepdims=True)
    d = h - mu
    var = jnp.mean(d * d, axis=1, keepdims=True)
    hn = d * lax.rsqrt(var + 1e-5) * g_ref[...] + b_ref[...]
    hs = hn * (1.0 / (1.0 + jnp.exp(-hn)))
    o = jnp.dot(hs.astype(jnp.bfloat16), w2_ref[...],
                preferred_element_type=jnp.float32)
    o_ref[...] = o + b2_ref[...]


def _mlp(x, W1, b1, ln_g, ln_b, W2, b2):
    grid = (N // MLP_BLK,)
    return pl.pallas_call(
        _mlp_body,
        grid=grid,
        in_specs=[
            pl.BlockSpec((MLP_BLK, IN_DIMS), lambda i: (i, 0)),
            pl.BlockSpec((IN_DIMS, HIDDEN), lambda i: (0, 0)),
            pl.BlockSpec((1, HIDDEN), lambda i: (0, 0)),
            pl.BlockSpec((1, HIDDEN), lambda i: (0, 0)),
            pl.BlockSpec((1, HIDDEN), lambda i: (0, 0)),
            pl.BlockSpec((HIDDEN, D), lambda i: (0, 0)),
            pl.BlockSpec((1, D), lambda i: (0, 0)),
        ],
        out_specs=pl.BlockSpec((MLP_BLK, D), lambda i: (i, 0)),
        out_shape=jax.ShapeDtypeStruct((N, D), jnp.float32),
        compiler_params=pltpu.CompilerParams(
            dimension_semantics=("arbitrary",),
        ),
    )(x, W1, b1, ln_g, ln_b, W2, b2)


_MESH = plsc.VectorSubcoreMesh(core_axis_name="c", subcore_axis_name="s")
_NEGINF = float("-inf")
_SC_PARAMS = pltpu.CompilerParams(needs_layout_passes=False,
                                  use_tc_tiling_on_sc=False)


@functools.partial(
    pl.kernel,
    mesh=_MESH,
    out_type=jax.ShapeDtypeStruct((AGGR_ROWS, D), jnp.float32),
    scratch_types=[
        pltpu.VMEM((CH, D), jnp.float32),
        pltpu.VMEM((CH, D), jnp.float32),
        pltpu.VMEM((CH,), jnp.int32),
        pltpu.VMEM((CH,), jnp.int32),
        pltpu.VMEM((CAP, D), jnp.float32),
        pltpu.VMEM((CAP * 16,), jnp.int32),
        pltpu.VMEM((8, 128), jnp.int32),
        pltpu.VMEM((16,), jnp.int32),
        pltpu.VMEM((16,), jnp.int32),
        pltpu.VMEM((4, 16), jnp.float32),
        pltpu.SemaphoreType.DMA,
        pltpu.SemaphoreType.DMA,
        pltpu.SemaphoreType.DMA,
        pltpu.SemaphoreType.DMA,
        pltpu.SemaphoreType.DMA,
    ],
    compiler_params=_SC_PARAMS,
)
def _scan(out_hbm, cl_hbm, aggr_hbm, data_a, data_b, cids_a, cids_b, stage_v,
          ids_v, idsc_v, pb_v, state_v, acc_v, sem, sem_da, sem_db, sem_ca,
          sem_cb):
    wid = lax.axis_index("s") * NC + lax.axis_index("c")
    base = wid * R
    end = base + R
    dummy = jnp.int32(C) + wid
    dvec = jnp.broadcast_to(dummy, (16,))
    neg = jnp.full((16,), _NEGINF, jnp.float32)
    iota = lax.iota(jnp.int32, 16)

    def _ids_reset(s, _):
        ids_v[pl.ds(s * 16, 16)] = dvec
        return 0

    lax.fori_loop(0, CAP, _ids_reset, 0)

    @pl.when(wid > 0)
    def _():
        pltpu.sync_copy(cl_hbm.at[pl.ds(base - 16, 16)], pb_v)

    prev0 = jnp.where(wid > 0, pb_v[...][15], jnp.int32(-1))

    def _scatter(epoch, run_id):
        @pl.when(epoch == 0)
        def _():
            ids_v[pl.ds(0, 16)] = dvec
        for j in range(CAP // 128):
            @pl.when(run_id >= j * 128)
            def _():
                for m in range(8):
                    g = plsc.load_gather(
                        ids_v, [(iota + (j * 128 + m * 16)) * 16])
                    idsc_v[j, pl.ds(m * 16, 16)] = g
                pltpu.async_copy(
                    stage_v.at[pl.ds(j * 128, 128)],
                    aggr_hbm.at[idsc_v.at[j]], sem).wait()

    def _flush(prev, run_id, epoch):
        fl = run_id >= FLUSH_THR

        @pl.when(fl)
        def _():
            _scatter(epoch, run_id)
            lax.fori_loop(0, CAP, _ids_reset, 0)
            for q in range(4):
                stage_v[0, pl.ds(q * 16, 16)] = stage_v[run_id,
                                                        pl.ds(q * 16, 16)]
            ids_v[pl.ds(0, 16)] = jnp.broadcast_to(prev, (16,))

        return jnp.where(fl, 0, run_id), epoch + fl.astype(jnp.int32)

    def _fire(k, db, cb, sd, sc):
        start = base + k * CH
        pltpu.async_copy(out_hbm.at[pl.ds(start, CH)], db, sd)
        pltpu.async_copy(cl_hbm.at[pl.ds(start, CH)], cb, sc)

    def _drain(db, cb, sd, sc):
        pltpu.make_async_copy(out_hbm.at[pl.ds(0, CH)], db, sd).wait()
        pltpu.make_async_copy(cl_hbm.at[pl.ds(0, CH)], cb, sc).wait()

    def _row_step(db, cid, r_slot, prev, run_id, accs):
        same = cid == prev
        run_id = run_id + (1 - same.astype(jnp.int32))
        for c in range(4):
            v = db[r_slot, pl.ds(c * 16, 16)]
            accs[c] = jnp.maximum(jnp.where(same, accs[c], neg), v)
            stage_v[run_id, pl.ds(c * 16, 16)] = accs[c]
        ids_v[pl.ds(run_id * 16, 16)] = jnp.broadcast_to(cid, (16,))
        return cid, run_id

    def _chunk(db, cb, sd, sc, carry):
        prev, run_id, epoch, a0, a1, a2, a3 = carry
        run_id, epoch = _flush(prev, run_id, epoch)
        _drain(db, cb, sd, sc)

        def _group(q, gc):
            prev, run_id, a0, a1, a2, a3 = gc
            cid_vec = cb[pl.ds(q * 16, 16)]
            ac = [a0, a1, a2, a3]
            for l in range(16):
                prev, run_id = _row_step(db, cid_vec[l], q * 16 + l, prev,
                                         run_id, ac)
            return (prev, run_id, ac[0], ac[1], ac[2], ac[3])

        prev, run_id, a0, a1, a2, a3 = lax.fori_loop(
            0, CH // 16, _group, (prev, run_id, a0, a1, a2, a3))
        return (prev, run_id, epoch, a0, a1, a2, a3)

    _fire(0, data_a, cids_a, sem_da, sem_ca)

    def _pair(j, carry):
        _fire(2 * j + 1, data_b, cids_b, sem_db, sem_cb)
        carry = _chunk(data_a, cids_a, sem_da, sem_ca, carry)
        _fire(2 * j + 2, data_a, cids_a, sem_da, sem_ca)
        carry = _chunk(data_b, cids_b, sem_db, sem_cb, carry)
        return carry

    carry = lax.fori_loop(0, (R // CH - 1) // 2, _pair,
                          (prev0, jnp.int32(0), jnp.int32(0),
                           neg, neg, neg, neg))
    prev, run_id, epoch, a0, a1, a2, a3 = _chunk(data_a, cids_a, sem_da,
                                                 sem_ca, carry)

    state_v[...] = jnp.where(iota == 0, prev,
                   jnp.where(iota == 1, run_id,
                   jnp.where(iota == 2, epoch, jnp.int32(0))))
    for c, a in enumerate((a0, a1, a2, a3)):
        acc_v[c, pl.ds(0, 16)] = a

    def _chunk_ext(j, _):
        st = state_v[...]

        @pl.when(st[3] == 0)
        def _():
            prev, run_id, epoch = st[0], st[1], st[2]
            run_id, epoch = _flush(prev, run_id, epoch)
            start = end + j * CH
            pltpu.sync_copy(out_hbm.at[pl.ds(start, CH)], data_a)
            pltpu.sync_copy(cl_hbm.at[pl.ds(start, CH)], cids_a)
            accs = [acc_v[c, pl.ds(0, 16)] for c in range(4)]

            def _group(q, gc):
                prev, run_id, done, a0, a1, a2, a3 = gc
                cid_vec = cids_a[pl.ds(q * 16, 16)]
                ac = [a0, a1, a2, a3]
                for l in range(16):
                    cid = cid_vec[l]
                    same = cid == prev
                    done_new = jnp.logical_or(done, jnp.logical_not(same))
                    adv = jnp.logical_and(jnp.logical_not(same),
                                          jnp.logical_not(done_new))
                    run_id = run_id + adv.astype(jnp.int32)
                    for c in range(4):
                        v = data_a[q * 16 + l, pl.ds(c * 16, 16)]
                        ac[c] = jnp.where(
                            done_new, ac[c],
                            jnp.maximum(jnp.where(same, ac[c], neg), v))
                        stage_v[run_id, pl.ds(c * 16, 16)] = ac[c]
                    idst = jnp.where(done_new, prev, cid)
                    ids_v[pl.ds(run_id * 16, 16)] = jnp.broadcast_to(idst,
                                                                     (16,))
                    prev = idst
                    done = done_new
                return (prev, run_id, done, ac[0], ac[1], ac[2], ac[3])

            prev, run_id, done, na0, na1, na2, na3 = lax.fori_loop(
                0, CH // 16, _group,
                (prev, run_id, jnp.bool_(False), accs[0], accs[1], accs[2],
                 accs[3]))
            state_v[...] = jnp.where(iota == 0, prev,
                           jnp.where(iota == 1, run_id,
                           jnp.where(iota == 2, epoch,
                                     done.astype(jnp.int32))))
            for c, a in enumerate((na0, na1, na2, na3)):
                acc_v[c, pl.ds(0, 16)] = a

        return 0

    lax.fori_loop(0, (N - end) // CH, _chunk_ext, 0)
    stf = state_v[...]
    _scatter(stf[2], stf[1])


GCH = 400
_G_PIECES = ((0, 128), (128, 128), (256, 128), (384, 16))


@functools.partial(
    pl.kernel,
    mesh=_MESH,
    out_type=jax.ShapeDtypeStruct((N, 2 * D), jnp.float32),
    scratch_types=[
        pltpu.VMEM((GCH,), jnp.int32),
        pltpu.VMEM((GCH,), jnp.int32),
        pltpu.VMEM((GCH, D), jnp.float32),
        pltpu.VMEM((GCH, D), jnp.float32),
        pltpu.VMEM((GCH, D), jnp.float32),
        pltpu.VMEM((GCH, D), jnp.float32),
        pltpu.SemaphoreType.DMA,
        pltpu.SemaphoreType.DMA,
        pltpu.SemaphoreType.DMA,
        pltpu.SemaphoreType.DMA,
        pltpu.SemaphoreType.DMA,
    ],
    compiler_params=_SC_PARAMS,
)
def _gather(cl_hbm, out_hbm, aggr_hbm, y_hbm, idx_a, idx_b, g_a, g_b, o_a,
            o_b, sem_ia, sem_ib, sem_oa, sem_ob, sem_g):
    wid = lax.axis_index("s") * NC + lax.axis_index("c")
    base = wid * R
    nch = R // GCH

    def _fire(k, idx_v, o_v, si, so):
        start = base + k * GCH
        pltpu.async_copy(cl_hbm.at[pl.ds(start, GCH)], idx_v, si)
        pltpu.async_copy(out_hbm.at[pl.ds(start, GCH)], o_v, so)

    def _process(k, idx_v, g_v, o_v, si, so):
        start = base + k * GCH
        pltpu.make_async_copy(cl_hbm.at[pl.ds(0, GCH)], idx_v, si).wait()
        cps = []
        for off, ln in _G_PIECES:
            cps.append(pltpu.async_copy(
                aggr_hbm.at[idx_v.at[pl.ds(off, ln)]],
                g_v.at[pl.ds(off, ln)], sem_g))
        pltpu.make_async_copy(out_hbm.at[pl.ds(0, GCH)], o_v, so).wait()
        pltpu.sync_copy(o_v, y_hbm.at[pl.ds(start, GCH), pl.ds(0, D)])
        for cp in cps:
            cp.wait()
        pltpu.sync_copy(g_v, y_hbm.at[pl.ds(start, GCH), pl.ds(D, D)])

    _fire(0, idx_a, o_a, sem_ia, sem_oa)

    def _pair(j, _):
        _fire(2 * j + 1, idx_b, o_b, sem_ib, sem_ob)
        _process(2 * j, idx_a, g_a, o_a, sem_ia, sem_oa)
        _fire(2 * j + 2, idx_a, o_a, sem_ia, sem_oa)
        _process(2 * j + 1, idx_b, g_b, o_b, sem_ib, sem_ob)
        return 0

    lax.fori_loop(0, (nch - 1) // 2, _pair, 0)
    _process(nch - 1, idx_a, g_a, o_a, sem_ia, sem_oa)


def kernel(x, cluster, batch, W1, b1, ln_g, ln_b, W2, b2):
    del batch
    cluster = cluster.astype(jnp.int32)
    out = _mlp(x, W1.astype(jnp.bfloat16), b1.reshape(1, -1),
               ln_g.reshape(1, -1), ln_b.reshape(1, -1),
               W2.astype(jnp.bfloat16), b2.reshape(1, -1))
    aggr = _scan(out, cluster)
    return _gather(cluster, out, aggr)


# speedup vs baseline: 2.3092x; 1.2404x over previous
import functools

import jax
import jax.numpy as jnp
from jax import lax
from jax.experimental import pallas as pl
from jax.experimental.pallas import tpu as pltpu
from jax.experimental.pallas import tpu_sc as plsc

N = 320000
IN_DIMS = 128
HIDDEN = 256
D = IN_DIMS // 2
C = 10000

NC = 2
NS = 16
NW = NC * NS
R = N // NW

CH = 400
CAP = 768
FLUSH_THR = CAP - CH - 1
AGGR_ROWS = C + NW

MLP_BLK = 1280


def _mlp_body(x_ref, w1_ref, b1_ref, g_ref, b_ref, w2_ref, b2_ref, o_ref):
    xb = x_ref[...].astype(jnp.bfloat16)
    h = jnp.dot(xb, w1_ref[...], preferred_element_type=jnp.float32)
    h = h + b1_ref[...]
    mu = jnp.mean(h, axis=1, keepdims=True)
    d = h - mu
    var = jnp.mean(d * d, axis=1, keepdims=True)
    hn = d * lax.rsqrt(var + 1e-5) * g_ref[...] + b_ref[...]
    hs = hn * (1.0 / (1.0 + jnp.exp(-hn)))
    o = jnp.dot(hs.astype(jnp.bfloat16), w2_ref[...],
                preferred_element_type=jnp.float32)
    o_ref[...] = o + b2_ref[...]


def _mlp(x, W1, b1, ln_g, ln_b, W2, b2):
    grid = (N // MLP_BLK,)
    return pl.pallas_call(
        _mlp_body,
        grid=grid,
        in_specs=[
            pl.BlockSpec((MLP_BLK, IN_DIMS), lambda i: (i, 0)),
            pl.BlockSpec((IN_DIMS, HIDDEN), lambda i: (0, 0)),
            pl.BlockSpec((1, HIDDEN), lambda i: (0, 0)),
            pl.BlockSpec((1, HIDDEN), lambda i: (0, 0)),
            pl.BlockSpec((1, HIDDEN), lambda i: (0, 0)),
            pl.BlockSpec((HIDDEN, D), lambda i: (0, 0)),
            pl.BlockSpec((1, D), lambda i: (0, 0)),
        ],
        out_specs=pl.BlockSpec((MLP_BLK, D), lambda i: (i, 0)),
        out_shape=jax.ShapeDtypeStruct((N, D), jnp.float32),
        compiler_params=pltpu.CompilerParams(
            dimension_semantics=("parallel",),
        ),
    )(x, W1, b1, ln_g, ln_b, W2, b2)


_MESH = plsc.VectorSubcoreMesh(core_axis_name="c", subcore_axis_name="s")
_NEGINF = float("-inf")
_SC_PARAMS = pltpu.CompilerParams(needs_layout_passes=False,
                                  use_tc_tiling_on_sc=False)


@functools.partial(
    pl.kernel,
    mesh=_MESH,
    out_type=jax.ShapeDtypeStruct((AGGR_ROWS, D), jnp.float32),
    scratch_types=[
        pltpu.VMEM((CH, D), jnp.float32),
        pltpu.VMEM((CH, D), jnp.float32),
        pltpu.VMEM((CH,), jnp.int32),
        pltpu.VMEM((CH,), jnp.int32),
        pltpu.VMEM((CAP, D), jnp.float32),
        pltpu.VMEM((CAP * 16,), jnp.int32),
        pltpu.VMEM((8, 128), jnp.int32),
        pltpu.VMEM((16,), jnp.int32),
        pltpu.VMEM((16,), jnp.int32),
        pltpu.VMEM((4, 16), jnp.float32),
        pltpu.SemaphoreType.DMA,
        pltpu.SemaphoreType.DMA,
        pltpu.SemaphoreType.DMA,
        pltpu.SemaphoreType.DMA,
        pltpu.SemaphoreType.DMA,
    ],
    compiler_params=_SC_PARAMS,
)
def _scan(out_hbm, cl_hbm, aggr_hbm, data_a, data_b, cids_a, cids_b, stage_v,
          ids_v, idsc_v, pb_v, state_v, acc_v, sem, sem_da, sem_db, sem_ca,
          sem_cb):
    wid = lax.axis_index("s") * NC + lax.axis_index("c")
    base = wid * R
    end = base + R
    dummy = jnp.int32(C) + wid
    dvec = jnp.broadcast_to(dummy, (16,))
    neg = jnp.full((16,), _NEGINF, jnp.float32)
    iota = lax.iota(jnp.int32, 16)

    def _ids_reset(s, _):
        ids_v[pl.ds(s * 16, 16)] = dvec
        return 0

    lax.fori_loop(0, CAP, _ids_reset, 0)

    @pl.when(wid > 0)
    def _():
        pltpu.sync_copy(cl_hbm.at[pl.ds(base - 16, 16)], pb_v)

    prev0 = jnp.where(wid > 0, pb_v[...][15], jnp.int32(-1))

    def _scatter(epoch, run_id):
        @pl.when(epoch == 0)
        def _():
            ids_v[pl.ds(0, 16)] = dvec
        for j in range(CAP // 128):
            @pl.when(run_id >= j * 128)
            def _():
                for m in range(8):
                    g = plsc.load_gather(
                        ids_v, [(iota + (j * 128 + m * 16)) * 16])
                    idsc_v[j, pl.ds(m * 16, 16)] = g
                pltpu.async_copy(
                    stage_v.at[pl.ds(j * 128, 128)],
                    aggr_hbm.at[idsc_v.at[j]], sem).wait()

    def _flush(prev, run_id, epoch):
        fl = run_id >= FLUSH_THR

        @pl.when(fl)
        def _():
            _scatter(epoch, run_id)
            lax.fori_loop(0, CAP, _ids_reset, 0)
            for q in range(4):
                stage_v[0, pl.ds(q * 16, 16)] = stage_v[run_id,
                                                        pl.ds(q * 16, 16)]
            ids_v[pl.ds(0, 16)] = jnp.broadcast_to(prev, (16,))

        return jnp.where(fl, 0, run_id), epoch + fl.astype(jnp.int32)

    def _fire(k, db, cb, sd, sc):
        start = base + k * CH
        pltpu.async_copy(out_hbm.at[pl.ds(start, CH)], db, sd)
        pltpu.async_copy(cl_hbm.at[pl.ds(start, CH)], cb, sc)

    def _drain(db, cb, sd, sc):
        pltpu.make_async_copy(out_hbm.at[pl.ds(0, CH)], db, sd).wait()
        pltpu.make_async_copy(cl_hbm.at[pl.ds(0, CH)], cb, sc).wait()

    def _row_step(db, cid, r_slot, prev, run_id, accs):
        same = cid == prev
        run_id = run_id + (1 - same.astype(jnp.int32))
        for c in range(4):
            v = db[r_slot, pl.ds(c * 16, 16)]
            accs[c] = jnp.maximum(jnp.where(same, accs[c], neg), v)
            stage_v[run_id, pl.ds(c * 16, 16)] = accs[c]
        ids_v[pl.ds(run_id * 16, 16)] = jnp.broadcast_to(cid, (16,))
        return cid, run_id

    def _chunk(db, cb, sd, sc, carry):
        prev, run_id, epoch, a0, a1, a2, a3 = carry
        run_id, epoch = _flush(prev, run_id, epoch)
        _drain(db, cb, sd, sc)

        def _group(q, gc):
            prev, run_id, a0, a1, a2, a3 = gc
            cid_vec = cb[pl.ds(q * 16, 16)]
            ac = [a0, a1, a2, a3]
            for l in range(16):
                prev, run_id = _row_step(db, cid_vec[l], q * 16 + l, prev,
                                         run_id, ac)
            return (prev, run_id, ac[0], ac[1], ac[2], ac[3])

        prev, run_id, a0, a1, a2, a3 = lax.fori_loop(
            0, CH // 16, _group, (prev, run_id, a0, a1, a2, a3))
        return (prev, run_id, epoch, a0, a1, a2, a3)

    _fire(0, data_a, cids_a, sem_da, sem_ca)

    def _pair(j, carry):
        _fire(2 * j + 1, data_b, cids_b, sem_db, sem_cb)
        carry = _chunk(data_a, cids_a, sem_da, sem_ca, carry)
        _fire(2 * j + 2, data_a, cids_a, sem_da, sem_ca)
        carry = _chunk(data_b, cids_b, sem_db, sem_cb, carry)
        return carry

    carry = lax.fori_loop(0, (R // CH - 1) // 2, _pair,
                          (prev0, jnp.int32(0), jnp.int32(0),
                           neg, neg, neg, neg))
    prev, run_id, epoch, a0, a1, a2, a3 = _chunk(data_a, cids_a, sem_da,
                                                 sem_ca, carry)

    state_v[...] = jnp.where(iota == 0, prev,
                   jnp.where(iota == 1, run_id,
                   jnp.where(iota == 2, epoch, jnp.int32(0))))
    for c, a in enumerate((a0, a1, a2, a3)):
        acc_v[c, pl.ds(0, 16)] = a

    def _chunk_ext(j, _):
        st = state_v[...]

        @pl.when(st[3] == 0)
        def _():
            prev, run_id, epoch = st[0], st[1], st[2]
            run_id, epoch = _flush(prev, run_id, epoch)
            start = end + j * CH
            pltpu.sync_copy(out_hbm.at[pl.ds(start, CH)], data_a)
            pltpu.sync_copy(cl_hbm.at[pl.ds(start, CH)], cids_a)
            accs = [acc_v[c, pl.ds(0, 16)] for c in range(4)]

            def _group(q, gc):
                prev, run_id, done, a0, a1, a2, a3 = gc
                cid_vec = cids_a[pl.ds(q * 16, 16)]
                ac = [a0, a1, a2, a3]
                for l in range(16):
                    cid = cid_vec[l]
                    same = cid == prev
                    done_new = jnp.logical_or(done, jnp.logical_not(same))
                    adv = jnp.logical_and(jnp.logical_not(same),
                                          jnp.logical_not(done_new))
                    run_id = run_id + adv.astype(jnp.int32)
                    for c in range(4):
                        v = data_a[q * 16 + l, pl.ds(c * 16, 16)]
                        ac[c] = jnp.where(
                            done_new, ac[c],
                            jnp.maximum(jnp.where(same, ac[c], neg), v))
                        stage_v[run_id, pl.ds(c * 16, 16)] = ac[c]
                    idst = jnp.where(done_new, prev, cid)
                    ids_v[pl.ds(run_id * 16, 16)] = jnp.broadcast_to(idst,
                                                                     (16,))
                    prev = idst
                    done = done_new
                return (prev, run_id, done, ac[0], ac[1], ac[2], ac[3])

            prev, run_id, done, na0, na1, na2, na3 = lax.fori_loop(
                0, CH // 16, _group,
                (prev, run_id, jnp.bool_(False), accs[0], accs[1], accs[2],
                 accs[3]))
            state_v[...] = jnp.where(iota == 0, prev,
                           jnp.where(iota == 1, run_id,
                           jnp.where(iota == 2, epoch,
                                     done.astype(jnp.int32))))
            for c, a in enumerate((na0, na1, na2, na3)):
                acc_v[c, pl.ds(0, 16)] = a

        return 0

    lax.fori_loop(0, (N - end) // CH, _chunk_ext, 0)
    stf = state_v[...]
    _scatter(stf[2], stf[1])


GCH = 400
_G_PIECES = ((0, 128), (128, 128), (256, 128), (384, 16))


@functools.partial(
    pl.kernel,
    mesh=_MESH,
    out_type=jax.ShapeDtypeStruct((N, 2 * D), jnp.float32),
    scratch_types=[
        pltpu.VMEM((GCH,), jnp.int32),
        pltpu.VMEM((GCH,), jnp.int32),
        pltpu.VMEM((GCH, D), jnp.float32),
        pltpu.VMEM((GCH, D), jnp.float32),
        pltpu.VMEM((GCH, D), jnp.float32),
        pltpu.VMEM((GCH, D), jnp.float32),
        pltpu.SemaphoreType.DMA,
        pltpu.SemaphoreType.DMA,
        pltpu.SemaphoreType.DMA,
        pltpu.SemaphoreType.DMA,
        pltpu.SemaphoreType.DMA,
    ],
    compiler_params=_SC_PARAMS,
)
def _gather(cl_hbm, out_hbm, aggr_hbm, y_hbm, idx_a, idx_b, g_a, g_b, o_a,
            o_b, sem_ia, sem_ib, sem_oa, sem_ob, sem_g):
    wid = lax.axis_index("s") * NC + lax.axis_index("c")
    base = wid * R
    nch = R // GCH

    def _fire(k, idx_v, o_v, si, so):
        start = base + k * GCH
        pltpu.async_copy(cl_hbm.at[pl.ds(start, GCH)], idx_v, si)
        pltpu.async_copy(out_hbm.at[pl.ds(start, GCH)], o_v, so)

    def _process(k, idx_v, g_v, o_v, si, so):
        start = base + k * GCH
        pltpu.make_async_copy(cl_hbm.at[pl.ds(0, GCH)], idx_v, si).wait()
        cps = []
        for off, ln in _G_PIECES:
            cps.append(pltpu.async_copy(
                aggr_hbm.at[idx_v.at[pl.ds(off, ln)]],
                g_v.at[pl.ds(off, ln)], sem_g))
        pltpu.make_async_copy(out_hbm.at[pl.ds(0, GCH)], o_v, so).wait()
        pltpu.sync_copy(o_v, y_hbm.at[pl.ds(start, GCH), pl.ds(0, D)])
        for cp in cps:
            cp.wait()
        pltpu.sync_copy(g_v, y_hbm.at[pl.ds(start, GCH), pl.ds(D, D)])

    _fire(0, idx_a, o_a, sem_ia, sem_oa)

    def _pair(j, _):
        _fire(2 * j + 1, idx_b, o_b, sem_ib, sem_ob)
        _process(2 * j, idx_a, g_a, o_a, sem_ia, sem_oa)
        _fire(2 * j + 2, idx_a, o_a, sem_ia, sem_oa)
        _process(2 * j + 1, idx_b, g_b, o_b, sem_ib, sem_ob)
        return 0

    lax.fori_loop(0, (nch - 1) // 2, _pair, 0)
    _process(nch - 1, idx_a, g_a, o_a, sem_ia, sem_oa)


def kernel(x, cluster, batch, W1, b1, ln_g, ln_b, W2, b2):
    del batch
    cluster = cluster.astype(jnp.int32)
    out = _mlp(x, W1.astype(jnp.bfloat16), b1.reshape(1, -1),
               ln_g.reshape(1, -1), ln_b.reshape(1, -1),
               W2.astype(jnp.bfloat16), b2.reshape(1, -1))
    aggr = _scan(out, cluster)
    return _gather(cluster, out, aggr)


# speedup vs baseline: 2.4529x; 1.0622x over previous
import functools

import jax
import jax.numpy as jnp
from jax import lax
from jax.experimental import pallas as pl
from jax.experimental.pallas import tpu as pltpu
from jax.experimental.pallas import tpu_sc as plsc

N = 320000
IN_DIMS = 128
HIDDEN = 256
D = IN_DIMS // 2
C = 10000

NC = 2
NS = 16
NW = NC * NS
R = N // NW

CH = 400
CAP = 768
FLUSH_THR = CAP - CH - 1
AGGR_ROWS = C + NW

MLP_BLK = 2000


def _mlp_body(x_ref, w1_ref, b1_ref, g_ref, b_ref, w2_ref, b2_ref, o_ref):
    xb = x_ref[...].astype(jnp.bfloat16)
    h = jnp.dot(xb, w1_ref[...], preferred_element_type=jnp.float32)
    h = h + b1_ref[...]
    mu = jnp.mean(h, axis=1, keepdims=True)
    d = h - mu
    var = jnp.mean(d * d, axis=1, keepdims=True)
    hn = d * lax.rsqrt(var + 1e-5) * g_ref[...] + b_ref[...]
    hs = hn * (1.0 / (1.0 + jnp.exp(-hn)))
    o = jnp.dot(hs.astype(jnp.bfloat16), w2_ref[...],
                preferred_element_type=jnp.float32)
    o_ref[...] = o + b2_ref[...]


def _mlp(x, W1, b1, ln_g, ln_b, W2, b2):
    grid = (N // MLP_BLK,)
    return pl.pallas_call(
        _mlp_body,
        grid=grid,
        in_specs=[
            pl.BlockSpec((MLP_BLK, IN_DIMS), lambda i: (i, 0)),
            pl.BlockSpec((IN_DIMS, HIDDEN), lambda i: (0, 0)),
            pl.BlockSpec((1, HIDDEN), lambda i: (0, 0)),
            pl.BlockSpec((1, HIDDEN), lambda i: (0, 0)),
            pl.BlockSpec((1, HIDDEN), lambda i: (0, 0)),
            pl.BlockSpec((HIDDEN, D), lambda i: (0, 0)),
            pl.BlockSpec((1, D), lambda i: (0, 0)),
        ],
        out_specs=pl.BlockSpec((MLP_BLK, D), lambda i: (i, 0)),
        out_shape=jax.ShapeDtypeStruct((N, D), jnp.float32),
        compiler_params=pltpu.CompilerParams(
            dimension_semantics=("parallel",),
        ),
    )(x, W1, b1, ln_g, ln_b, W2, b2)


_MESH = plsc.VectorSubcoreMesh(core_axis_name="c", subcore_axis_name="s")
_NEGINF = float("-inf")
_SC_PARAMS = pltpu.CompilerParams(needs_layout_passes=False,
                                  use_tc_tiling_on_sc=False)


@functools.partial(
    pl.kernel,
    mesh=_MESH,
    out_type=jax.ShapeDtypeStruct((AGGR_ROWS, D), jnp.float32),
    scratch_types=[
        pltpu.VMEM((CH, D), jnp.float32),
        pltpu.VMEM((CH, D), jnp.float32),
        pltpu.VMEM((CH,), jnp.int32),
        pltpu.VMEM((CH,), jnp.int32),
        pltpu.VMEM((CAP, D), jnp.float32),
        pltpu.VMEM((CAP * 16,), jnp.int32),
        pltpu.VMEM((8, 128), jnp.int32),
        pltpu.VMEM((16,), jnp.int32),
        pltpu.VMEM((16,), jnp.int32),
        pltpu.VMEM((4, 16), jnp.float32),
        pltpu.SemaphoreType.DMA,
        pltpu.SemaphoreType.DMA,
        pltpu.SemaphoreType.DMA,
        pltpu.SemaphoreType.DMA,
        pltpu.SemaphoreType.DMA,
    ],
    compiler_params=_SC_PARAMS,
)
def _scan(out_hbm, cl_hbm, aggr_hbm, data_a, data_b, cids_a, cids_b, stage_v,
          ids_v, idsc_v, pb_v, state_v, acc_v, sem, sem_da, sem_db, sem_ca,
          sem_cb):
    wid = lax.axis_index("s") * NC + lax.axis_index("c")
    base = wid * R
    end = base + R
    dummy = jnp.int32(C) + wid
    dvec = jnp.broadcast_to(dummy, (16,))
    neg = jnp.full((16,), _NEGINF, jnp.float32)
    iota = lax.iota(jnp.int32, 16)

    def _ids_reset(s, _):
        ids_v[pl.ds(s * 16, 16)] = dvec
        return 0

    lax.fori_loop(0, CAP, _ids_reset, 0)

    @pl.when(wid > 0)
    def _():
        pltpu.sync_copy(cl_hbm.at[pl.ds(base - 16, 16)], pb_v)

    prev0 = jnp.where(wid > 0, pb_v[...][15], jnp.int32(-1))

    def _scatter(epoch, run_id):
        @pl.when(epoch == 0)
        def _():
            ids_v[pl.ds(0, 16)] = dvec
        for j in range(CAP // 128):
            @pl.when(run_id >= j * 128)
            def _():
                for m in range(8):
                    g = plsc.load_gather(
                        ids_v, [(iota + (j * 128 + m * 16)) * 16])
                    idsc_v[j, pl.ds(m * 16, 16)] = g
                pltpu.async_copy(
                    stage_v.at[pl.ds(j * 128, 128)],
                    aggr_hbm.at[idsc_v.at[j]], sem).wait()

    def _flush(prev, run_id, epoch):
        fl = run_id >= FLUSH_THR

        @pl.when(fl)
        def _():
            _scatter(epoch, run_id)
            lax.fori_loop(0, CAP, _ids_reset, 0)
            for q in range(4):
                stage_v[0, pl.ds(q * 16, 16)] = stage_v[run_id,
                                                        pl.ds(q * 16, 16)]
            ids_v[pl.ds(0, 16)] = jnp.broadcast_to(prev, (16,))

        return jnp.where(fl, 0, run_id), epoch + fl.astype(jnp.int32)

    def _fire(k, db, cb, sd, sc):
        start = base + k * CH
        pltpu.async_copy(out_hbm.at[pl.ds(start, CH)], db, sd)
        pltpu.async_copy(cl_hbm.at[pl.ds(start, CH)], cb, sc)

    def _drain(db, cb, sd, sc):
        pltpu.make_async_copy(out_hbm.at[pl.ds(0, CH)], db, sd).wait()
        pltpu.make_async_copy(cl_hbm.at[pl.ds(0, CH)], cb, sc).wait()

    def _row_step(db, cid, r_slot, prev, run_id, accs):
        same = cid == prev
        run_id = run_id + (1 - same.astype(jnp.int32))
        for c in range(4):
            v = db[r_slot, pl.ds(c * 16, 16)]
            accs[c] = jnp.maximum(jnp.where(same, accs[c], neg), v)
            stage_v[run_id, pl.ds(c * 16, 16)] = accs[c]
        ids_v[pl.ds(run_id * 16, 16)] = jnp.broadcast_to(cid, (16,))
        return cid, run_id

    def _chunk(db, cb, sd, sc, carry):
        prev, run_id, epoch, a0, a1, a2, a3 = carry
        run_id, epoch = _flush(prev, run_id, epoch)
        _drain(db, cb, sd, sc)

        def _group(q, gc):
            prev, run_id, a0, a1, a2, a3 = gc
            cid_vec = cb[pl.ds(q * 16, 16)]
            ac = [a0, a1, a2, a3]
            for l in range(16):
                prev, run_id = _row_step(db, cid_vec[l], q * 16 + l, prev,
                                         run_id, ac)
            return (prev, run_id, ac[0], ac[1], ac[2], ac[3])

        prev, run_id, a0, a1, a2, a3 = lax.fori_loop(
            0, CH // 16, _group, (prev, run_id, a0, a1, a2, a3))
        return (prev, run_id, epoch, a0, a1, a2, a3)

    _fire(0, data_a, cids_a, sem_da, sem_ca)

    def _pair(j, carry):
        _fire(2 * j + 1, data_b, cids_b, sem_db, sem_cb)
        carry = _chunk(data_a, cids_a, sem_da, sem_ca, carry)
        _fire(2 * j + 2, data_a, cids_a, sem_da, sem_ca)
        carry = _chunk(data_b, cids_b, sem_db, sem_cb, carry)
        return carry

    carry = lax.fori_loop(0, (R // CH - 1) // 2, _pair,
                          (prev0, jnp.int32(0), jnp.int32(0),
                           neg, neg, neg, neg))
    prev, run_id, epoch, a0, a1, a2, a3 = _chunk(data_a, cids_a, sem_da,
                                                 sem_ca, carry)

    state_v[...] = jnp.where(iota == 0, prev,
                   jnp.where(iota == 1, run_id,
                   jnp.where(iota == 2, epoch, jnp.int32(0))))
    for c, a in enumerate((a0, a1, a2, a3)):
        acc_v[c, pl.ds(0, 16)] = a

    def _chunk_ext(j, _):
        st = state_v[...]

        @pl.when(st[3] == 0)
        def _():
            prev, run_id, epoch = st[0], st[1], st[2]
            run_id, epoch = _flush(prev, run_id, epoch)
            start = end + j * CH
            pltpu.sync_copy(out_hbm.at[pl.ds(start, CH)], data_a)
            pltpu.sync_copy(cl_hbm.at[pl.ds(start, CH)], cids_a)
            accs = [acc_v[c, pl.ds(0, 16)] for c in range(4)]

            def _group(q, gc):
                prev, run_id, done, a0, a1, a2, a3 = gc
                cid_vec = cids_a[pl.ds(q * 16, 16)]
                ac = [a0, a1, a2, a3]
                for l in range(16):
                    cid = cid_vec[l]
                    same = cid == prev
                    done_new = jnp.logical_or(done, jnp.logical_not(same))
                    adv = jnp.logical_and(jnp.logical_not(same),
                                          jnp.logical_not(done_new))
                    run_id = run_id + adv.astype(jnp.int32)
                    for c in range(4):
                        v = data_a[q * 16 + l, pl.ds(c * 16, 16)]
                        ac[c] = jnp.where(
                            done_new, ac[c],
                            jnp.maximum(jnp.where(same, ac[c], neg), v))
                        stage_v[run_id, pl.ds(c * 16, 16)] = ac[c]
                    idst = jnp.where(done_new, prev, cid)
                    ids_v[pl.ds(run_id * 16, 16)] = jnp.broadcast_to(idst,
                                                                     (16,))
                    prev = idst
                    done = done_new
                return (prev, run_id, done, ac[0], ac[1], ac[2], ac[3])

            prev, run_id, done, na0, na1, na2, na3 = lax.fori_loop(
                0, CH // 16, _group,
                (prev, run_id, jnp.bool_(False), accs[0], accs[1], accs[2],
                 accs[3]))
            state_v[...] = jnp.where(iota == 0, prev,
                           jnp.where(iota == 1, run_id,
                           jnp.where(iota == 2, epoch,
                                     done.astype(jnp.int32))))
            for c, a in enumerate((na0, na1, na2, na3)):
                acc_v[c, pl.ds(0, 16)] = a

        return 0

    lax.fori_loop(0, (N - end) // CH, _chunk_ext, 0)
    stf = state_v[...]
    _scatter(stf[2], stf[1])


GCH = 400
_G_PIECES = ((0, 128), (128, 128), (256, 128), (384, 16))


@functools.partial(
    pl.kernel,
    mesh=_MESH,
    out_type=jax.ShapeDtypeStruct((N, 2 * D), jnp.float32),
    scratch_types=[
        pltpu.VMEM((GCH,), jnp.int32),
        pltpu.VMEM((GCH,), jnp.int32),
        pltpu.VMEM((GCH, D), jnp.float32),
        pltpu.VMEM((GCH, D), jnp.float32),
        pltpu.VMEM((GCH, D), jnp.float32),
        pltpu.VMEM((GCH, D), jnp.float32),
        pltpu.SemaphoreType.DMA,
        pltpu.SemaphoreType.DMA,
        pltpu.SemaphoreType.DMA,
        pltpu.SemaphoreType.DMA,
        pltpu.SemaphoreType.DMA,
        pltpu.SemaphoreType.DMA,
        pltpu.SemaphoreType.DMA,
    ],
    compiler_params=_SC_PARAMS,
)
def _gather(cl_hbm, out_hbm, aggr_hbm, y_hbm, idx_a, idx_b, g_a, g_b, o_a,
            o_b, sem_ia, sem_ib, sem_oa, sem_ob, sem_g, sem_wa, sem_wb):
    wid = lax.axis_index("s") * NC + lax.axis_index("c")
    base = wid * R
    nch = R // GCH

    def _fire(k, idx_v, o_v, si, so):
        start = base + k * GCH
        pltpu.async_copy(cl_hbm.at[pl.ds(start, GCH)], idx_v, si)
        pltpu.async_copy(out_hbm.at[pl.ds(start, GCH)], o_v, so)

    def _process(k, idx_v, g_v, o_v, si, so, sw):
        start = base + k * GCH
        pltpu.make_async_copy(cl_hbm.at[pl.ds(0, GCH)], idx_v, si).wait()
        cps = []
        for off, ln in _G_PIECES:
            cps.append(pltpu.async_copy(
                aggr_hbm.at[idx_v.at[pl.ds(off, ln)]],
                g_v.at[pl.ds(off, ln)], sem_g))
        pltpu.make_async_copy(out_hbm.at[pl.ds(0, GCH)], o_v, so).wait()
        pltpu.async_copy(o_v, y_hbm.at[pl.ds(start, GCH), pl.ds(0, D)], sw)
        for cp in cps:
            cp.wait()
        pltpu.async_copy(g_v, y_hbm.at[pl.ds(start, GCH), pl.ds(D, D)], sw)

    def _drain_w(g_v, o_v, sw):
        pltpu.make_async_copy(o_v, y_hbm.at[pl.ds(0, GCH), pl.ds(0, D)],
                              sw).wait()
        pltpu.make_async_copy(g_v, y_hbm.at[pl.ds(0, GCH), pl.ds(D, D)],
                              sw).wait()

    _fire(0, idx_a, o_a, sem_ia, sem_oa)

    def _pair(j, _):
        @pl.when(j > 0)
        def _():
            _drain_w(g_b, o_b, sem_wb)
        _fire(2 * j + 1, idx_b, o_b, sem_ib, sem_ob)
        _process(2 * j, idx_a, g_a, o_a, sem_ia, sem_oa, sem_wa)
        _drain_w(g_a, o_a, sem_wa)
        _fire(2 * j + 2, idx_a, o_a, sem_ia, sem_oa)
        _process(2 * j + 1, idx_b, g_b, o_b, sem_ib, sem_ob, sem_wb)
        return 0

    lax.fori_loop(0, (nch - 1) // 2, _pair, 0)
    _drain_w(g_b, o_b, sem_wb)
    _process(nch - 1, idx_a, g_a, o_a, sem_ia, sem_oa, sem_wa)
    _drain_w(g_a, o_a, sem_wa)


def kernel(x, cluster, batch, W1, b1, ln_g, ln_b, W2, b2):
    del batch
    cluster = cluster.astype(jnp.int32)
    out = _mlp(x, W1.astype(jnp.bfloat16), b1.reshape(1, -1),
               ln_g.reshape(1, -1), ln_b.reshape(1, -1),
               W2.astype(jnp.bfloat16), b2.reshape(1, -1))
    aggr = _scan(out, cluster)
    return _gather(cluster, out, aggr)


# speedup vs baseline: 2.5034x; 1.0206x over previous
import functools

import jax
import jax.numpy as jnp
from jax import lax
from jax.experimental import pallas as pl
from jax.experimental.pallas import tpu as pltpu
from jax.experimental.pallas import tpu_sc as plsc

N = 320000
IN_DIMS = 128
HIDDEN = 256
D = IN_DIMS // 2
C = 10000

NC = 2
NS = 16
NW = NC * NS
R = N // NW

CH = 400
CAP = 768
FLUSH_THR = CAP - CH - 1
AGGR_ROWS = C + NW

MLP_BLK = 2000


def _mlp_body(x_ref, w1_ref, b1_ref, g_ref, b_ref, w2_ref, b2_ref, o_ref):
    xb = x_ref[...].astype(jnp.bfloat16)
    h = jnp.dot(xb, w1_ref[...], preferred_element_type=jnp.float32)
    h = h + b1_ref[...]
    mu = jnp.mean(h, axis=1, keepdims=True)
    d = h - mu
    var = jnp.mean(d * d, axis=1, keepdims=True)
    hn = d * lax.rsqrt(var + 1e-5) * g_ref[...] + b_ref[...]
    hs = hn * (1.0 / (1.0 + jnp.exp(-hn)))
    o = jnp.dot(hs.astype(jnp.bfloat16), w2_ref[...],
                preferred_element_type=jnp.float32)
    o_ref[...] = o + b2_ref[...]


def _mlp(x, W1, b1, ln_g, ln_b, W2, b2):
    grid = (N // MLP_BLK,)
    return pl.pallas_call(
        _mlp_body,
        grid=grid,
        in_specs=[
            pl.BlockSpec((MLP_BLK, IN_DIMS), lambda i: (i, 0)),
            pl.BlockSpec((IN_DIMS, HIDDEN), lambda i: (0, 0)),
            pl.BlockSpec((1, HIDDEN), lambda i: (0, 0)),
            pl.BlockSpec((1, HIDDEN), lambda i: (0, 0)),
            pl.BlockSpec((1, HIDDEN), lambda i: (0, 0)),
            pl.BlockSpec((HIDDEN, D), lambda i: (0, 0)),
            pl.BlockSpec((1, D), lambda i: (0, 0)),
        ],
        out_specs=pl.BlockSpec((MLP_BLK, D), lambda i: (i, 0)),
        out_shape=jax.ShapeDtypeStruct((N, D), jnp.float32),
        compiler_params=pltpu.CompilerParams(
            dimension_semantics=("parallel",),
        ),
    )(x, W1, b1, ln_g, ln_b, W2, b2)


_MESH = plsc.VectorSubcoreMesh(core_axis_name="c", subcore_axis_name="s")
_NEGINF = float("-inf")
_SC_PARAMS = pltpu.CompilerParams(needs_layout_passes=False,
                                  use_tc_tiling_on_sc=False)


@functools.partial(
    pl.kernel,
    mesh=_MESH,
    out_type=jax.ShapeDtypeStruct((AGGR_ROWS, D), jnp.float32),
    scratch_types=[
        pltpu.VMEM((CH, D), jnp.float32),
        pltpu.VMEM((CH, D), jnp.float32),
        pltpu.VMEM((CH,), jnp.int32),
        pltpu.VMEM((CH,), jnp.int32),
        pltpu.VMEM((CAP, D), jnp.float32),
        pltpu.VMEM((CAP * 16,), jnp.int32),
        pltpu.VMEM((8, 128), jnp.int32),
        pltpu.VMEM((16,), jnp.int32),
        pltpu.VMEM((16,), jnp.int32),
        pltpu.VMEM((4, 16), jnp.float32),
        pltpu.SemaphoreType.DMA,
        pltpu.SemaphoreType.DMA,
        pltpu.SemaphoreType.DMA,
        pltpu.SemaphoreType.DMA,
        pltpu.SemaphoreType.DMA,
    ],
    compiler_params=_SC_PARAMS,
)
def _scan(out_hbm, cl_hbm, aggr_hbm, data_a, data_b, cids_a, cids_b, stage_v,
          ids_v, idsc_v, pb_v, state_v, acc_v, sem, sem_da, sem_db, sem_ca,
          sem_cb):
    wid = lax.axis_index("s") * NC + lax.axis_index("c")
    base = wid * R
    end = base + R
    dummy = jnp.int32(C) + wid
    dvec = jnp.broadcast_to(dummy, (16,))
    neg = jnp.full((16,), _NEGINF, jnp.float32)
    iota = lax.iota(jnp.int32, 16)

    def _ids_reset(s, _):
        ids_v[pl.ds(s * 16, 16)] = dvec
        return 0

    lax.fori_loop(0, CAP, _ids_reset, 0)

    @pl.when(wid > 0)
    def _():
        pltpu.sync_copy(cl_hbm.at[pl.ds(base - 16, 16)], pb_v)

    prev0 = jnp.where(wid > 0, pb_v[...][15], jnp.int32(-1))

    def _scatter(epoch, run_id):
        @pl.when(epoch == 0)
        def _():
            ids_v[pl.ds(0, 16)] = dvec
        for j in range(CAP // 128):
            @pl.when(run_id >= j * 128)
            def _():
                for m in range(8):
                    g = plsc.load_gather(
                        ids_v, [(iota + (j * 128 + m * 16)) * 16])
                    idsc_v[j, pl.ds(m * 16, 16)] = g
                pltpu.async_copy(
                    stage_v.at[pl.ds(j * 128, 128)],
                    aggr_hbm.at[idsc_v.at[j]], sem).wait()

    def _flush(prev, run_id, epoch):
        fl = run_id >= FLUSH_THR

        @pl.when(fl)
        def _():
            _scatter(epoch, run_id)
            lax.fori_loop(0, CAP, _ids_reset, 0)
            for q in range(4):
                stage_v[0, pl.ds(q * 16, 16)] = stage_v[run_id,
                                                        pl.ds(q * 16, 16)]
            ids_v[pl.ds(0, 16)] = jnp.broadcast_to(prev, (16,))

        return jnp.where(fl, 0, run_id), epoch + fl.astype(jnp.int32)

    def _fire(k, db, cb, sd, sc):
        start = base + k * CH
        pltpu.async_copy(out_hbm.at[pl.ds(start, CH)], db, sd)
        pltpu.async_copy(cl_hbm.at[pl.ds(start, CH)], cb, sc)

    def _drain(db, cb, sd, sc):
        pltpu.make_async_copy(out_hbm.at[pl.ds(0, CH)], db, sd).wait()
        pltpu.make_async_copy(cl_hbm.at[pl.ds(0, CH)], cb, sc).wait()

    def _row_step(db, cid, r_slot, prev, run_id, accs):
        same = cid == prev
        run_id = run_id + (1 - same.astype(jnp.int32))
        for c in range(4):
            v = db[r_slot, pl.ds(c * 16, 16)]
            accs[c] = jnp.maximum(jnp.where(same, accs[c], neg), v)
            stage_v[run_id, pl.ds(c * 16, 16)] = accs[c]
        ids_v[pl.ds(run_id * 16, 16)] = jnp.broadcast_to(cid, (16,))
        return cid, run_id

    def _chunk(db, cb, sd, sc, carry):
        prev, run_id, epoch, a0, a1, a2, a3 = carry
        run_id, epoch = _flush(prev, run_id, epoch)
        _drain(db, cb, sd, sc)

        def _group(q, gc):
            prev, run_id, a0, a1, a2, a3 = gc
            cid_vec = cb[pl.ds(q * 16, 16)]
            ac = [a0, a1, a2, a3]
            for l in range(16):
                prev, run_id = _row_step(db, cid_vec[l], q * 16 + l, prev,
                                         run_id, ac)
            return (prev, run_id, ac[0], ac[1], ac[2], ac[3])

        prev, run_id, a0, a1, a2, a3 = lax.fori_loop(
            0, CH // 16, _group, (prev, run_id, a0, a1, a2, a3))
        return (prev, run_id, epoch, a0, a1, a2, a3)

    _fire(0, data_a, cids_a, sem_da, sem_ca)

    def _pair(j, carry):
        _fire(2 * j + 1, data_b, cids_b, sem_db, sem_cb)
        carry = _chunk(data_a, cids_a, sem_da, sem_ca, carry)
        _fire(2 * j + 2, data_a, cids_a, sem_da, sem_ca)
        carry = _chunk(data_b, cids_b, sem_db, sem_cb, carry)
        return carry

    carry = lax.fori_loop(0, (R // CH - 1) // 2, _pair,
                          (prev0, jnp.int32(0), jnp.int32(0),
                           neg, neg, neg, neg))
    prev, run_id, epoch, a0, a1, a2, a3 = _chunk(data_a, cids_a, sem_da,
                                                 sem_ca, carry)

    state_v[...] = jnp.where(iota == 0, prev,
                   jnp.where(iota == 1, run_id,
                   jnp.where(iota == 2, epoch, jnp.int32(0))))
    for c, a in enumerate((a0, a1, a2, a3)):
        acc_v[c, pl.ds(0, 16)] = a

    def _chunk_ext(j, _):
        st = state_v[...]

        @pl.when(st[3] == 0)
        def _():
            prev, run_id, epoch = st[0], st[1], st[2]
            run_id, epoch = _flush(prev, run_id, epoch)
            start = end + j * CH
            pltpu.sync_copy(out_hbm.at[pl.ds(start, CH)], data_a)
            pltpu.sync_copy(cl_hbm.at[pl.ds(start, CH)], cids_a)
            accs = [acc_v[c, pl.ds(0, 16)] for c in range(4)]

            def _group(q, gc):
                prev, run_id, done, a0, a1, a2, a3 = gc
                cid_vec = cids_a[pl.ds(q * 16, 16)]
                ac = [a0, a1, a2, a3]
                for l in range(16):
                    cid = cid_vec[l]
                    same = cid == prev
                    done_new = jnp.logical_or(done, jnp.logical_not(same))
                    adv = jnp.logical_and(jnp.logical_not(same),
                                          jnp.logical_not(done_new))
                    run_id = run_id + adv.astype(jnp.int32)
                    for c in range(4):
                        v = data_a[q * 16 + l, pl.ds(c * 16, 16)]
                        ac[c] = jnp.where(
                            done_new, ac[c],
                            jnp.maximum(jnp.where(same, ac[c], neg), v))
                        stage_v[run_id, pl.ds(c * 16, 16)] = ac[c]
                    idst = jnp.where(done_new, prev, cid)
                    ids_v[pl.ds(run_id * 16, 16)] = jnp.broadcast_to(idst,
                                                                     (16,))
                    prev = idst
                    done = done_new
                return (prev, run_id, done, ac[0], ac[1], ac[2], ac[3])

            prev, run_id, done, na0, na1, na2, na3 = lax.fori_loop(
                0, CH // 16, _group,
                (prev, run_id, jnp.bool_(False), accs[0], accs[1], accs[2],
                 accs[3]))
            state_v[...] = jnp.where(iota == 0, prev,
                           jnp.where(iota == 1, run_id,
                           jnp.where(iota == 2, epoch,
                                     done.astype(jnp.int32))))
            for c, a in enumerate((na0, na1, na2, na3)):
                acc_v[c, pl.ds(0, 16)] = a

        return 0

    lax.fori_loop(0, (N - end) // CH, _chunk_ext, 0)
    stf = state_v[...]
    _scatter(stf[2], stf[1])


GCH = 200
NCHG = R // GCH
_G_PIECES = ((0, 128), (128, 72))
_NBUF = 3


@functools.partial(
    pl.kernel,
    mesh=_MESH,
    out_type=jax.ShapeDtypeStruct((N, 2 * D), jnp.float32),
    scratch_types=(
        [pltpu.VMEM((GCH,), jnp.int32) for _ in range(_NBUF)] +
        [pltpu.VMEM((GCH, D), jnp.float32) for _ in range(_NBUF)] +
        [pltpu.VMEM((GCH, D), jnp.float32) for _ in range(_NBUF)] +
        [pltpu.SemaphoreType.DMA for _ in range(4 * _NBUF)]
    ),
    compiler_params=_SC_PARAMS,
)
def _gather(cl_hbm, out_hbm, aggr_hbm, y_hbm, *bufs):
    idx_b = bufs[0:_NBUF]
    g_b = bufs[_NBUF:2 * _NBUF]
    o_b = bufs[2 * _NBUF:3 * _NBUF]
    sems = bufs[3 * _NBUF:]
    sem_i = sems[0:_NBUF]
    sem_o = sems[_NBUF:2 * _NBUF]
    sem_g = sems[2 * _NBUF:3 * _NBUF]
    sem_w = sems[3 * _NBUF:4 * _NBUF]

    wid = lax.axis_index("s") * NC + lax.axis_index("c")
    base = wid * R

    ld_i = [None] * NCHG
    ld_o = [None] * NCHG
    gts = [None] * NCHG
    wrs = [None] * NCHG

    def fire_loads(k):
        b = k % _NBUF
        start = base + k * GCH
        ld_i[k] = pltpu.async_copy(cl_hbm.at[pl.ds(start, GCH)], idx_b[b],
                                   sem_i[b])
        ld_o[k] = pltpu.async_copy(out_hbm.at[pl.ds(start, GCH)], o_b[b],
                                   sem_o[b])

    def fire_gathers(k):
        b = k % _NBUF
        ld_i[k].wait()
        gts[k] = [pltpu.async_copy(
            aggr_hbm.at[idx_b[b].at[pl.ds(off, ln)]],
            g_b[b].at[pl.ds(off, ln)], sem_g[b]) for off, ln in _G_PIECES]

    def fire_writes(k):
        b = k % _NBUF
        start = base + k * GCH
        ld_o[k].wait()
        w1 = pltpu.async_copy(o_b[b], y_hbm.at[pl.ds(start, GCH),
                                               pl.ds(0, D)], sem_w[b])
        for cp in gts[k]:
            cp.wait()
        w2 = pltpu.async_copy(g_b[b], y_hbm.at[pl.ds(start, GCH),
                                               pl.ds(D, D)], sem_w[b])
        wrs[k] = (w1, w2)

    for k in range(NCHG + 2):
        if k < NCHG:
            if k >= _NBUF:
                for cp in wrs[k - _NBUF]:
                    cp.wait()
            fire_loads(k)
        if 0 <= k - 1 < NCHG:
            fire_gathers(k - 1)
        if 0 <= k - 2 < NCHG:
            fire_writes(k - 2)
    for k in range(NCHG - _NBUF, NCHG):
        for cp in wrs[k]:
            cp.wait()


def kernel(x, cluster, batch, W1, b1, ln_g, ln_b, W2, b2):
    del batch
    cluster = cluster.astype(jnp.int32)
    out = _mlp(x, W1.astype(jnp.bfloat16), b1.reshape(1, -1),
               ln_g.reshape(1, -1), ln_b.reshape(1, -1),
               W2.astype(jnp.bfloat16), b2.reshape(1, -1))
    aggr = _scan(out, cluster)
    return _gather(cluster, out, aggr)
